# baseline (speedup 1.0000x reference)
import dataclasses
import functools

import jax
import jax.numpy as jnp
from jax import lax
from jax.experimental import pallas as pl
from jax.experimental.pallas import tpu as pltpu
from jax.experimental.pallas import tpu_sc as plsc

_N = 10000
_E = 320000
_D = 128
_DE = 16
_G = 128

_NC = 2
_NS = 16
_NW = _NC * _NS
_L = 16

_CP = pltpu.CompilerParams()
if "needs_layout_passes" in pltpu.CompilerParams.__dataclass_fields__:
    _CP = dataclasses.replace(_CP, needs_layout_passes=False)

_EPW = _E // _NW
_EB = 80
_NP = 10240
_RPS = _NP // _NS
_ZB = 128

_BE = 2000
_BN = 2000


def _edge_e_body(ea_ref, we_ref, o_ref):
    o_ref[...] = jnp.dot(ea_ref[...], we_ref[...],
                         preferred_element_type=jnp.float32)


def _edge_e(edge_attr, W_e):
    return pl.pallas_call(
        _edge_e_body,
        grid=(_E // _BE,),
        in_specs=[
            pl.BlockSpec((_BE, _DE), lambda i: (i, 0)),
            pl.BlockSpec((_DE, _D), lambda i: (0, 0)),
        ],
        out_specs=pl.BlockSpec((_BE, _D), lambda i: (i, 0)),
        out_shape=jax.ShapeDtypeStruct((_E, _D), jnp.float32),
    )(edge_attr, W_e)


def _edge_m_body(hs_ref, e_ref, w_ref, b_ref, o_ref):
    s = hs_ref[...] + e_ref[...]
    o_ref[...] = jnp.maximum(
        jnp.dot(s, w_ref[...], preferred_element_type=jnp.float32)
        + b_ref[...], 0.0)


def _edge_m(hs, e, W, b2d):
    return pl.pallas_call(
        _edge_m_body,
        grid=(_E // _BE,),
        in_specs=[
            pl.BlockSpec((_BE, _D), lambda i: (i, 0)),
            pl.BlockSpec((_BE, _D), lambda i: (i, 0)),
            pl.BlockSpec((_D, _D), lambda i: (0, 0)),
            pl.BlockSpec((1, _D), lambda i: (0, 0)),
        ],
        out_specs=pl.BlockSpec((_BE, _D), lambda i: (i, 0)),
        out_shape=jax.ShapeDtypeStruct((_E, _D), jnp.float32),
    )(hs, e, W, b2d)


def _node_upd_body(h_ref, agg_ref, u_ref, o_ref):
    o_ref[...] = h_ref[...] + jnp.dot(
        agg_ref[...], u_ref[...], preferred_element_type=jnp.float32)


def _node_upd(h, agg, U):
    return pl.pallas_call(
        _node_upd_body,
        grid=(_N // _BN,),
        in_specs=[
            pl.BlockSpec((_BN, _D), lambda i: (i, 0)),
            pl.BlockSpec((_BN, _D), lambda i: (i, 0)),
            pl.BlockSpec((_D, _D), lambda i: (0, 0)),
        ],
        out_specs=pl.BlockSpec((_BN, _D), lambda i: (i, 0)),
        out_shape=jax.ShapeDtypeStruct((_N, _D), jnp.float32),
    )(h, agg, U)


def _final_body(h_ref, agg_ref, u_ref, ids_ref, lng_ref, lnb_ref, wh1_ref,
                bh1_ref, bng_ref, bnb_ref, wh2_ref, bh2_ref, out_ref,
                pool_ref):
    i = pl.program_id(0)

    @pl.when(i == 0)
    def _():
        pool_ref[...] = jnp.zeros_like(pool_ref)

    h2 = h_ref[...] + jnp.dot(
        agg_ref[...], u_ref[...], preferred_element_type=jnp.float32)
    ids = ids_ref[0]
    gids = lax.broadcasted_iota(jnp.int32, (_G, 1), 0)
    mask = (ids == gids).astype(jnp.float32)
    pool_ref[...] += jnp.dot(mask, h2, preferred_element_type=jnp.float32,
                             precision=lax.Precision.HIGHEST)

    @pl.when(i == pl.num_programs(0) - 1)
    def _():
        g = pool_ref[...]
        mu = jnp.mean(g, axis=-1, keepdims=True)
        var = jnp.mean((g - mu) ** 2, axis=-1, keepdims=True)
        z = (g - mu) * lax.rsqrt(var + 1e-5) * lng_ref[...] + lnb_ref[...]
        z = jnp.dot(z, wh1_ref[...], preferred_element_type=jnp.float32) + bh1_ref[...]
        bmu = jnp.mean(z, axis=0, keepdims=True)
        bvar = jnp.mean((z - bmu) ** 2, axis=0, keepdims=True)
        z = (z - bmu) * lax.rsqrt(bvar + 1e-5) * bng_ref[...] + bnb_ref[...]
        z = jnp.maximum(z, 0.0)
        o = jnp.dot(z, wh2_ref[...], preferred_element_type=jnp.float32) + bh2_ref[...]
        out_ref[...] = jnp.clip(o, 0.0, 100.0)


def _final(h1, agg, U2, ids3d, ln_g, ln_b, Wh1, bh1, bn_g, bn_b, Wh2, bh2):
    out, _ = pl.pallas_call(
        _final_body,
        grid=(_N // _BN,),
        in_specs=[
            pl.BlockSpec((_BN, _D), lambda i: (i, 0)),
            pl.BlockSpec((_BN, _D), lambda i: (i, 0)),
            pl.BlockSpec((_D, _D), lambda i: (0, 0)),
            pl.BlockSpec((1, 1, _BN), lambda i: (i, 0, 0)),
            pl.BlockSpec((1, _D), lambda i: (0, 0)),
            pl.BlockSpec((1, _D), lambda i: (0, 0)),
            pl.BlockSpec((_D, _D), lambda i: (0, 0)),
            pl.BlockSpec((1, _D), lambda i: (0, 0)),
            pl.BlockSpec((1, _D), lambda i: (0, 0)),
            pl.BlockSpec((1, _D), lambda i: (0, 0)),
            pl.BlockSpec((_D, 1), lambda i: (0, 0)),
            pl.BlockSpec((1, 1), lambda i: (0, 0)),
        ],
        out_specs=[
            pl.BlockSpec((_G, 1), lambda i: (0, 0)),
            pl.BlockSpec((_G, _D), lambda i: (0, 0)),
        ],
        out_shape=[
            jax.ShapeDtypeStruct((_G, 1), jnp.float32),
            jax.ShapeDtypeStruct((_G, _D), jnp.float32),
        ],
    )(h1, agg, U2, ids3d, ln_g, ln_b, Wh1, bh1, bn_g, bn_b, Wh2, bh2)
    return out


def _sc_gather_impl(h_hbm, src_hbm, out_hbm, idx_v, rows_v, sem):
    cid = lax.axis_index("c")
    sid = lax.axis_index("s")
    wid = sid * _NC + cid
    base_w = wid * _EPW

    @pl.loop(0, _EPW, step=_EB)
    def _(eb):
        base = base_w + eb
        pltpu.sync_copy(src_hbm.at[pl.ds(base, _EB)], idx_v)
        pltpu.async_copy(h_hbm.at[idx_v], rows_v, sem).wait()
        pltpu.sync_copy(rows_v, out_hbm.at[pl.ds(base, _EB)])


@functools.cache
def _make_sc_gather():
    return pl.kernel(
        _sc_gather_impl,
        out_type=jax.ShapeDtypeStruct((_E, _D), jnp.float32),
        mesh=plsc.VectorSubcoreMesh(core_axis_name="c", subcore_axis_name="s"),
        scratch_types=[
            pltpu.VMEM((_EB,), jnp.int32),
            pltpu.VMEM((_EB, _D), jnp.float32),
            pltpu.SemaphoreType.DMA,
        ],
    )


def _sc_gather(h, src):
    return _make_sc_gather()(h, src)


_RPW = _NP // _NW
_SB = 2000
_CAP = 16384
_PB = 80


def _sc_scatter_impl(m_hbm, dst_hbm, out_hbm, blkA_v, blkB_v, idxl_v, dstl_v,
                     bufA_v, bufB_v, acc_v, pos_s, semSA, semSB, semA, semB):
    cid = lax.axis_index("c")
    sid = lax.axis_index("s")
    wid = sid * _NC + cid
    lo = wid * _RPW

    @pl.loop(0, _RPW)
    def _(r):
        @pl.loop(0, _D, step=_L)
        def _(cc):
            acc_v.at[r, pl.ds(cc, _L)][...] = jnp.zeros((_L,), jnp.float32)

    pos_s[0] = 0

    nblk = _E // _SB

    def _scan_block(eb, blk_v):
        @pl.loop(0, _SB, step=_L)
        def _(c0):
            d = blk_v[pl.ds(c0, _L)]
            msk = (d >= lo) & (d < lo + _RPW)
            mi = msk.astype(jnp.int32)
            cnt = jnp.sum(mi)

            @pl.when(cnt > 0)
            def _():
                pos = pos_s[0]
                positions = pos + plsc.cumsum(mi) - 1
                eidx = eb + c0 + lax.iota(jnp.int32, _L)
                plsc.store_scatter(idxl_v, [positions], eidx, mask=msk)
                plsc.store_scatter(dstl_v, [positions], d - lo, mask=msk)
                pos_s[0] = pos + cnt

    pltpu.async_copy(dst_hbm.at[pl.ds(0, _SB)], blkA_v, semSA)

    @pl.loop(0, nblk)
    def _(k):
        @pl.when(k % 2 == 0)
        def _():
            pltpu.make_async_copy(dst_hbm.at[pl.ds(k * _SB, _SB)],
                                  blkA_v, semSA).wait()

            @pl.when(k + 1 < nblk)
            def _():
                pltpu.async_copy(dst_hbm.at[pl.ds((k + 1) * _SB, _SB)],
                                 blkB_v, semSB)
            _scan_block(k * _SB, blkA_v)

        @pl.when(k % 2 == 1)
        def _():
            pltpu.make_async_copy(dst_hbm.at[pl.ds(k * _SB, _SB)],
                                  blkB_v, semSB).wait()

            @pl.when(k + 1 < nblk)
            def _():
                pltpu.async_copy(dst_hbm.at[pl.ds((k + 1) * _SB, _SB)],
                                 blkA_v, semSA)
            _scan_block(k * _SB, blkB_v)

    total = pos_s[0]

    @pl.loop(0, _PB, step=_L)
    def _(t):
        plsc.store_scatter(idxl_v, [total + t + lax.iota(jnp.int32, _L)],
                           jnp.zeros((_L,), jnp.int32))

    nb = (total + _PB - 1) // _PB

    def _gather_start(j, buf_v, sem):
        pltpu.async_copy(m_hbm.at[idxl_v.at[pl.ds(j * _PB, _PB)]], buf_v, sem)

    def _gather_wait(j, buf_v, sem):
        pltpu.make_async_copy(m_hbm.at[idxl_v.at[pl.ds(j * _PB, _PB)]],
                              buf_v, sem).wait()

    def _apply(j, buf_v):
        @pl.loop(0, _PB, step=_L)
        def _(b0):
            for l in range(_L):
                valid = jnp.broadcast_to(j * _PB + b0 + l < total, (_L,))
                dsplat = plsc.load_gather(
                    dstl_v, [jnp.broadcast_to(j * _PB + b0 + l, (_L,))])
                for c in range(_D // _L):
                    col = lax.iota(jnp.int32, _L) + c * _L
                    v = buf_v[b0 + l, pl.ds(c * _L, _L)]
                    plsc.addupdate_scatter(acc_v, [dsplat, col], v, mask=valid)

    @pl.when(nb > 0)
    def _():
        _gather_start(0, bufA_v, semA)

    @pl.loop(0, nb)
    def _(j):
        @pl.when(j % 2 == 0)
        def _():
            _gather_wait(j, bufA_v, semA)

            @pl.when(j + 1 < nb)
            def _():
                _gather_start(j + 1, bufB_v, semB)
            _apply(j, bufA_v)

        @pl.when(j % 2 == 1)
        def _():
            _gather_wait(j, bufB_v, semB)

            @pl.when(j + 1 < nb)
            def _():
                _gather_start(j + 1, bufA_v, semA)
            _apply(j, bufB_v)

    pltpu.sync_copy(acc_v, out_hbm.at[pl.ds(lo, _RPW)])


@functools.cache
def _make_sc_scatter():
    return pl.kernel(
        _sc_scatter_impl,
        out_type=jax.ShapeDtypeStruct((_NP, _D), jnp.float32),
        mesh=plsc.VectorSubcoreMesh(core_axis_name="c", subcore_axis_name="s"),
        compiler_params=_CP,
        scratch_types=[
            pltpu.VMEM((_SB,), jnp.int32),
            pltpu.VMEM((_SB,), jnp.int32),
            pltpu.VMEM((_CAP,), jnp.int32),
            pltpu.VMEM((_CAP,), jnp.int32),
            pltpu.VMEM((_PB, _D), jnp.float32),
            pltpu.VMEM((_PB, _D), jnp.float32),
            pltpu.VMEM((_RPW, _D), jnp.float32),
            pltpu.SMEM((1,), jnp.int32),
            pltpu.SemaphoreType.DMA,
            pltpu.SemaphoreType.DMA,
            pltpu.SemaphoreType.DMA,
            pltpu.SemaphoreType.DMA,
        ],
    )


def _sc_scatter(m, dst):
    return _make_sc_scatter()(m, dst)


_PBND = (0, 720, 1440, 2160, 2880, 3600, 4320, 5040, 5760, 6480, 7200,
         7680, 8160, 8640, 9120, 9600, 10000)
_NPC = len(_PBND) - 1
_PRB = 80


def _sc_pool_impl(h_hbm, ids_hbm, out_hbm, ids_v, rows_v, acc_v, sem):
    cid = lax.axis_index("c")
    sid = lax.axis_index("s")
    wid = sid * _NC + cid

    @pl.when(wid < _NPC)
    def _():
        lo80 = jnp.int32(0)
        hi80 = jnp.int32(0)
        for j in range(_NPC):
            lo80 = jnp.where(wid == j, jnp.int32(_PBND[j] // _PRB), lo80)
            hi80 = jnp.where(wid == j, jnp.int32(_PBND[j + 1] // _PRB), hi80)

        @pl.loop(0, _G)
        def _(r):
            @pl.loop(0, _D, step=_L)
            def _(cc):
                acc_v.at[r, pl.ds(cc, _L)][...] = jnp.zeros(
                    (_L,), jnp.float32)

        nblk = hi80 - lo80

        @pl.loop(0, nblk)
        def _(k):
            base = (lo80 + k) * _PRB
            pltpu.sync_copy(ids_hbm.at[pl.ds(base, _PRB)], ids_v)
            pltpu.sync_copy(h_hbm.at[pl.ds(base, _PRB)], rows_v)

            @pl.loop(0, _PRB)
            def _(r):
                gsplat = plsc.load_gather(ids_v, [jnp.broadcast_to(r, (_L,))])
                for c in range(_D // _L):
                    col = lax.iota(jnp.int32, _L) + c * _L
                    v = rows_v[r, pl.ds(c * _L, _L)]
                    plsc.addupdate_scatter(acc_v, [gsplat, col], v)

        pltpu.sync_copy(acc_v, out_hbm.at[wid])


@functools.cache
def _make_sc_pool():
    return pl.kernel(
        _sc_pool_impl,
        out_type=jax.ShapeDtypeStruct((_NPC, _G, _D), jnp.float32),
        mesh=plsc.VectorSubcoreMesh(core_axis_name="c", subcore_axis_name="s"),
        compiler_params=_CP,
        scratch_types=[
            pltpu.VMEM((_PRB,), jnp.int32),
            pltpu.VMEM((_PRB, _D), jnp.float32),
            pltpu.VMEM((_G, _D), jnp.float32),
            pltpu.SemaphoreType.DMA,
        ],
    )


def _sc_pool(h2, ids):
    return _make_sc_pool()(h2, ids)


def _head_body(parts_ref, lng_ref, lnb_ref, wh1_ref, bh1_ref, bng_ref,
               bnb_ref, wh2_ref, bh2_ref, out_ref):
    g = parts_ref[0]
    for w in range(1, _NPC):
        g = g + parts_ref[w]
    mu = jnp.mean(g, axis=-1, keepdims=True)
    var = jnp.mean((g - mu) ** 2, axis=-1, keepdims=True)
    z = (g - mu) * lax.rsqrt(var + 1e-5) * lng_ref[...] + lnb_ref[...]
    z = jnp.dot(z, wh1_ref[...], preferred_element_type=jnp.float32) + bh1_ref[...]
    bmu = jnp.mean(z, axis=0, keepdims=True)
    bvar = jnp.mean((z - bmu) ** 2, axis=0, keepdims=True)
    z = (z - bmu) * lax.rsqrt(bvar + 1e-5) * bng_ref[...] + bnb_ref[...]
    z = jnp.maximum(z, 0.0)
    o = jnp.dot(z, wh2_ref[...], preferred_element_type=jnp.float32) + bh2_ref[...]
    out_ref[...] = jnp.clip(o, 0.0, 100.0)


def _head(parts, ln_g, ln_b, Wh1, bh1, bn_g, bn_b, Wh2, bh2):
    return pl.pallas_call(
        _head_body,
        out_shape=jax.ShapeDtypeStruct((_G, 1), jnp.float32),
    )(parts, ln_g, ln_b, Wh1, bh1, bn_g, bn_b, Wh2, bh2)


def kernel(x, edge_index, edge_attr, node_graph_ids, W_e, W1, b1, U1, W2, b2,
           U2, ln_g, ln_b, Wh1, bh1, bn_g, bn_b, Wh2, bh2):
    src = edge_index[0]
    dst = edge_index[1]
    b1r = b1.reshape(1, _D)
    b2r = b2.reshape(1, _D)

    e = _edge_e(edge_attr, W_e)

    hs1 = _sc_gather(x, src)
    m1 = _edge_m(hs1, e, W1, b1r)
    agg1 = _sc_scatter(m1, dst)
    h1 = _node_upd(x, agg1, U1)

    hs2 = _sc_gather(h1, src)
    m2 = _edge_m(hs2, e, W2, b2r)
    agg2 = _sc_scatter(m2, dst)
    h2 = _node_upd(h1, agg2, U2)

    parts = _sc_pool(h2, node_graph_ids)
    out = _head(parts, ln_g.reshape(1, _D), ln_b.reshape(1, _D),
                Wh1, bh1.reshape(1, _D), bn_g.reshape(1, _D),
                bn_b.reshape(1, _D), Wh2, bh2.reshape(1, 1))
    return out[:, 0]


# speedup vs baseline: 1.1673x; 1.1673x over previous
import dataclasses
import functools

import jax
import jax.numpy as jnp
from jax import lax
from jax.experimental import pallas as pl
from jax.experimental.pallas import tpu as pltpu
from jax.experimental.pallas import tpu_sc as plsc

_N = 10000
_E = 320000
_D = 128
_DE = 16
_G = 128

_NC = 2
_NS = 16
_NW = _NC * _NS
_L = 16

_CP = pltpu.CompilerParams()
if "needs_layout_passes" in pltpu.CompilerParams.__dataclass_fields__:
    _CP = dataclasses.replace(_CP, needs_layout_passes=False)

_EPW = _E // _NW
_EB = 80
_NP = 10240
_RPS = _NP // _NS
_ZB = 128

_BE = 2000
_BN = 2000


def _edge_e_body(ea_ref, we_ref, o_ref):
    o_ref[...] = jnp.dot(ea_ref[...], we_ref[...],
                         preferred_element_type=jnp.float32)


def _edge_e(edge_attr, W_e):
    return pl.pallas_call(
        _edge_e_body,
        grid=(_E // _BE,),
        in_specs=[
            pl.BlockSpec((_BE, _DE), lambda i: (i, 0)),
            pl.BlockSpec((_DE, _D), lambda i: (0, 0)),
        ],
        out_specs=pl.BlockSpec((_BE, _D), lambda i: (i, 0)),
        out_shape=jax.ShapeDtypeStruct((_E, _D), jnp.float32),
    )(edge_attr, W_e)


def _edge_m_body(hs_ref, e_ref, w_ref, b_ref, o_ref):
    s = hs_ref[...] + e_ref[...]
    o_ref[...] = jnp.maximum(
        jnp.dot(s, w_ref[...], preferred_element_type=jnp.float32)
        + b_ref[...], 0.0)


def _edge_m(hs, e, W, b2d):
    return pl.pallas_call(
        _edge_m_body,
        grid=(_E // _BE,),
        in_specs=[
            pl.BlockSpec((_BE, _D), lambda i: (i, 0)),
            pl.BlockSpec((_BE, _D), lambda i: (i, 0)),
            pl.BlockSpec((_D, _D), lambda i: (0, 0)),
            pl.BlockSpec((1, _D), lambda i: (0, 0)),
        ],
        out_specs=pl.BlockSpec((_BE, _D), lambda i: (i, 0)),
        out_shape=jax.ShapeDtypeStruct((_E, _D), jnp.float32),
    )(hs, e, W, b2d)


def _node_upd_body(h_ref, agg_ref, u_ref, o_ref):
    o_ref[...] = h_ref[...] + jnp.dot(
        agg_ref[...], u_ref[...], preferred_element_type=jnp.float32)


def _node_upd(h, agg, U):
    return pl.pallas_call(
        _node_upd_body,
        grid=(_N // _BN,),
        in_specs=[
            pl.BlockSpec((_BN, _D), lambda i: (i, 0)),
            pl.BlockSpec((_BN, _D), lambda i: (i, 0)),
            pl.BlockSpec((_D, _D), lambda i: (0, 0)),
        ],
        out_specs=pl.BlockSpec((_BN, _D), lambda i: (i, 0)),
        out_shape=jax.ShapeDtypeStruct((_N, _D), jnp.float32),
    )(h, agg, U)


def _final_body(h_ref, agg_ref, u_ref, ids_ref, lng_ref, lnb_ref, wh1_ref,
                bh1_ref, bng_ref, bnb_ref, wh2_ref, bh2_ref, out_ref,
                pool_ref):
    i = pl.program_id(0)

    @pl.when(i == 0)
    def _():
        pool_ref[...] = jnp.zeros_like(pool_ref)

    h2 = h_ref[...] + jnp.dot(
        agg_ref[...], u_ref[...], preferred_element_type=jnp.float32)
    ids = ids_ref[0]
    gids = lax.broadcasted_iota(jnp.int32, (_G, 1), 0)
    mask = (ids == gids).astype(jnp.float32)
    pool_ref[...] += jnp.dot(mask, h2, preferred_element_type=jnp.float32,
                             precision=lax.Precision.HIGHEST)

    @pl.when(i == pl.num_programs(0) - 1)
    def _():
        g = pool_ref[...]
        mu = jnp.mean(g, axis=-1, keepdims=True)
        var = jnp.mean((g - mu) ** 2, axis=-1, keepdims=True)
        z = (g - mu) * lax.rsqrt(var + 1e-5) * lng_ref[...] + lnb_ref[...]
        z = jnp.dot(z, wh1_ref[...], preferred_element_type=jnp.float32) + bh1_ref[...]
        bmu = jnp.mean(z, axis=0, keepdims=True)
        bvar = jnp.mean((z - bmu) ** 2, axis=0, keepdims=True)
        z = (z - bmu) * lax.rsqrt(bvar + 1e-5) * bng_ref[...] + bnb_ref[...]
        z = jnp.maximum(z, 0.0)
        o = jnp.dot(z, wh2_ref[...], preferred_element_type=jnp.float32) + bh2_ref[...]
        out_ref[...] = jnp.clip(o, 0.0, 100.0)


def _final(h1, agg, U2, ids3d, ln_g, ln_b, Wh1, bh1, bn_g, bn_b, Wh2, bh2):
    out, _ = pl.pallas_call(
        _final_body,
        grid=(_N // _BN,),
        in_specs=[
            pl.BlockSpec((_BN, _D), lambda i: (i, 0)),
            pl.BlockSpec((_BN, _D), lambda i: (i, 0)),
            pl.BlockSpec((_D, _D), lambda i: (0, 0)),
            pl.BlockSpec((1, 1, _BN), lambda i: (i, 0, 0)),
            pl.BlockSpec((1, _D), lambda i: (0, 0)),
            pl.BlockSpec((1, _D), lambda i: (0, 0)),
            pl.BlockSpec((_D, _D), lambda i: (0, 0)),
            pl.BlockSpec((1, _D), lambda i: (0, 0)),
            pl.BlockSpec((1, _D), lambda i: (0, 0)),
            pl.BlockSpec((1, _D), lambda i: (0, 0)),
            pl.BlockSpec((_D, 1), lambda i: (0, 0)),
            pl.BlockSpec((1, 1), lambda i: (0, 0)),
        ],
        out_specs=[
            pl.BlockSpec((_G, 1), lambda i: (0, 0)),
            pl.BlockSpec((_G, _D), lambda i: (0, 0)),
        ],
        out_shape=[
            jax.ShapeDtypeStruct((_G, 1), jnp.float32),
            jax.ShapeDtypeStruct((_G, _D), jnp.float32),
        ],
    )(h1, agg, U2, ids3d, ln_g, ln_b, Wh1, bh1, bn_g, bn_b, Wh2, bh2)
    return out


def _sc_gather_impl(h_hbm, src_hbm, out_hbm, idx_v, rows_v, sem):
    cid = lax.axis_index("c")
    sid = lax.axis_index("s")
    wid = sid * _NC + cid
    base_w = wid * _EPW

    @pl.loop(0, _EPW, step=_EB)
    def _(eb):
        base = base_w + eb
        pltpu.sync_copy(src_hbm.at[pl.ds(base, _EB)], idx_v)
        pltpu.async_copy(h_hbm.at[idx_v], rows_v, sem).wait()
        pltpu.sync_copy(rows_v, out_hbm.at[pl.ds(base, _EB)])


@functools.cache
def _make_sc_gather():
    return pl.kernel(
        _sc_gather_impl,
        out_type=jax.ShapeDtypeStruct((_E, _D), jnp.float32),
        mesh=plsc.VectorSubcoreMesh(core_axis_name="c", subcore_axis_name="s"),
        scratch_types=[
            pltpu.VMEM((_EB,), jnp.int32),
            pltpu.VMEM((_EB, _D), jnp.float32),
            pltpu.SemaphoreType.DMA,
        ],
    )


def _sc_gather(h, src):
    return _make_sc_gather()(h, src)


_RPW = _NP // _NW
_SB = 2000
_CAP = 16384
_PB = 128


def _sc_scatter_impl(m_hbm, dst_hbm, out_hbm, blkA_v, blkB_v, idxl_v, dstl_v,
                     bufA_v, bufB_v, acc_v, pos_s, semSA, semSB, semA, semB):
    cid = lax.axis_index("c")
    sid = lax.axis_index("s")
    wid = sid * _NC + cid
    lo = wid * _RPW

    @pl.loop(0, _RPW)
    def _(r):
        @pl.loop(0, _D, step=_L)
        def _(cc):
            acc_v.at[r, pl.ds(cc, _L)][...] = jnp.zeros((_L,), jnp.float32)

    pos_s[0] = 0

    nblk = _E // _SB

    def _scan_block(eb, blk_v):
        @pl.loop(0, _SB, step=2 * _L)
        def _(c0):
            d0 = blk_v[pl.ds(c0, _L)]
            d1 = blk_v[pl.ds(c0 + _L, _L)]
            msk0 = (d0 >= lo) & (d0 < lo + _RPW)
            msk1 = (d1 >= lo) & (d1 < lo + _RPW)
            mi0 = msk0.astype(jnp.int32)
            mi1 = msk1.astype(jnp.int32)
            cnt0 = jnp.sum(mi0)
            cnt1 = jnp.sum(mi1)

            @pl.when(cnt0 + cnt1 > 0)
            def _():
                pos = pos_s[0]
                pref0 = plsc.cumsum(mi0)
                pref1 = plsc.cumsum(mi1)
                positions0 = pos + pref0 - 1
                positions1 = pos + cnt0 + pref1 - 1
                iot = lax.iota(jnp.int32, _L)
                plsc.store_scatter(idxl_v, [positions0], eb + c0 + iot,
                                   mask=msk0)
                plsc.store_scatter(dstl_v, [positions0], d0 - lo, mask=msk0)
                plsc.store_scatter(idxl_v, [positions1], eb + c0 + _L + iot,
                                   mask=msk1)
                plsc.store_scatter(dstl_v, [positions1], d1 - lo, mask=msk1)
                pos_s[0] = pos + cnt0 + cnt1

    pltpu.async_copy(dst_hbm.at[pl.ds(0, _SB)], blkA_v, semSA)

    @pl.loop(0, nblk)
    def _(k):
        @pl.when(k % 2 == 0)
        def _():
            pltpu.make_async_copy(dst_hbm.at[pl.ds(k * _SB, _SB)],
                                  blkA_v, semSA).wait()

            @pl.when(k + 1 < nblk)
            def _():
                pltpu.async_copy(dst_hbm.at[pl.ds((k + 1) * _SB, _SB)],
                                 blkB_v, semSB)
            _scan_block(k * _SB, blkA_v)

        @pl.when(k % 2 == 1)
        def _():
            pltpu.make_async_copy(dst_hbm.at[pl.ds(k * _SB, _SB)],
                                  blkB_v, semSB).wait()

            @pl.when(k + 1 < nblk)
            def _():
                pltpu.async_copy(dst_hbm.at[pl.ds((k + 1) * _SB, _SB)],
                                 blkA_v, semSA)
            _scan_block(k * _SB, blkB_v)

    total = pos_s[0]

    @pl.loop(0, _PB, step=_L)
    def _(t):
        plsc.store_scatter(idxl_v, [total + t + lax.iota(jnp.int32, _L)],
                           jnp.zeros((_L,), jnp.int32))

    nb = (total + _PB - 1) // _PB

    def _gather_start(j, buf_v, sem):
        pltpu.async_copy(m_hbm.at[idxl_v.at[pl.ds(j * _PB, _PB)]], buf_v, sem)

    def _gather_wait(j, buf_v, sem):
        pltpu.make_async_copy(m_hbm.at[idxl_v.at[pl.ds(j * _PB, _PB)]],
                              buf_v, sem).wait()

    def _apply(j, buf_v):
        @pl.loop(0, _PB, step=_L)
        def _(b0):
            for l in range(_L):
                valid = jnp.broadcast_to(j * _PB + b0 + l < total, (_L,))
                dsplat = plsc.load_gather(
                    dstl_v, [jnp.broadcast_to(j * _PB + b0 + l, (_L,))])
                for c in range(_D // _L):
                    col = lax.iota(jnp.int32, _L) + c * _L
                    v = buf_v[b0 + l, pl.ds(c * _L, _L)]
                    plsc.addupdate_scatter(acc_v, [dsplat, col], v, mask=valid)

    @pl.when(nb > 0)
    def _():
        _gather_start(0, bufA_v, semA)

    @pl.loop(0, nb)
    def _(j):
        @pl.when(j % 2 == 0)
        def _():
            _gather_wait(j, bufA_v, semA)

            @pl.when(j + 1 < nb)
            def _():
                _gather_start(j + 1, bufB_v, semB)
            _apply(j, bufA_v)

        @pl.when(j % 2 == 1)
        def _():
            _gather_wait(j, bufB_v, semB)

            @pl.when(j + 1 < nb)
            def _():
                _gather_start(j + 1, bufA_v, semA)
            _apply(j, bufB_v)

    pltpu.sync_copy(acc_v, out_hbm.at[pl.ds(lo, _RPW)])


@functools.cache
def _make_sc_scatter():
    return pl.kernel(
        _sc_scatter_impl,
        out_type=jax.ShapeDtypeStruct((_NP, _D), jnp.float32),
        mesh=plsc.VectorSubcoreMesh(core_axis_name="c", subcore_axis_name="s"),
        compiler_params=_CP,
        scratch_types=[
            pltpu.VMEM((_SB,), jnp.int32),
            pltpu.VMEM((_SB,), jnp.int32),
            pltpu.VMEM((_CAP,), jnp.int32),
            pltpu.VMEM((_CAP,), jnp.int32),
            pltpu.VMEM((_PB, _D), jnp.float32),
            pltpu.VMEM((_PB, _D), jnp.float32),
            pltpu.VMEM((_RPW, _D), jnp.float32),
            pltpu.SMEM((1,), jnp.int32),
            pltpu.SemaphoreType.DMA,
            pltpu.SemaphoreType.DMA,
            pltpu.SemaphoreType.DMA,
            pltpu.SemaphoreType.DMA,
        ],
    )


def _sc_scatter(m, dst):
    return _make_sc_scatter()(m, dst)


_PBND = (0, 720, 1440, 2160, 2880, 3600, 4320, 5040, 5760, 6480, 7200,
         7680, 8160, 8640, 9120, 9600, 10000)
_NPC = len(_PBND) - 1
_PRB = 80


def _sc_pool_impl(h_hbm, ids_hbm, out_hbm, ids_v, rows_v, acc_v, sem):
    cid = lax.axis_index("c")
    sid = lax.axis_index("s")
    wid = sid * _NC + cid

    @pl.when(wid < _NPC)
    def _():
        lo80 = jnp.int32(0)
        hi80 = jnp.int32(0)
        for j in range(_NPC):
            lo80 = jnp.where(wid == j, jnp.int32(_PBND[j] // _PRB), lo80)
            hi80 = jnp.where(wid == j, jnp.int32(_PBND[j + 1] // _PRB), hi80)

        @pl.loop(0, _G)
        def _(r):
            @pl.loop(0, _D, step=_L)
            def _(cc):
                acc_v.at[r, pl.ds(cc, _L)][...] = jnp.zeros(
                    (_L,), jnp.float32)

        nblk = hi80 - lo80

        @pl.loop(0, nblk)
        def _(k):
            base = (lo80 + k) * _PRB
            pltpu.sync_copy(ids_hbm.at[pl.ds(base, _PRB)], ids_v)
            pltpu.sync_copy(h_hbm.at[pl.ds(base, _PRB)], rows_v)

            @pl.loop(0, _PRB)
            def _(r):
                gsplat = plsc.load_gather(ids_v, [jnp.broadcast_to(r, (_L,))])
                for c in range(_D // _L):
                    col = lax.iota(jnp.int32, _L) + c * _L
                    v = rows_v[r, pl.ds(c * _L, _L)]
                    plsc.addupdate_scatter(acc_v, [gsplat, col], v)

        pltpu.sync_copy(acc_v, out_hbm.at[wid])


@functools.cache
def _make_sc_pool():
    return pl.kernel(
        _sc_pool_impl,
        out_type=jax.ShapeDtypeStruct((_NPC, _G, _D), jnp.float32),
        mesh=plsc.VectorSubcoreMesh(core_axis_name="c", subcore_axis_name="s"),
        compiler_params=_CP,
        scratch_types=[
            pltpu.VMEM((_PRB,), jnp.int32),
            pltpu.VMEM((_PRB, _D), jnp.float32),
            pltpu.VMEM((_G, _D), jnp.float32),
            pltpu.SemaphoreType.DMA,
        ],
    )


def _sc_pool(h2, ids):
    return _make_sc_pool()(h2, ids)


def _head_body(parts_ref, lng_ref, lnb_ref, wh1_ref, bh1_ref, bng_ref,
               bnb_ref, wh2_ref, bh2_ref, out_ref):
    g = parts_ref[0]
    for w in range(1, _NPC):
        g = g + parts_ref[w]
    mu = jnp.mean(g, axis=-1, keepdims=True)
    var = jnp.mean((g - mu) ** 2, axis=-1, keepdims=True)
    z = (g - mu) * lax.rsqrt(var + 1e-5) * lng_ref[...] + lnb_ref[...]
    z = jnp.dot(z, wh1_ref[...], preferred_element_type=jnp.float32) + bh1_ref[...]
    bmu = jnp.mean(z, axis=0, keepdims=True)
    bvar = jnp.mean((z - bmu) ** 2, axis=0, keepdims=True)
    z = (z - bmu) * lax.rsqrt(bvar + 1e-5) * bng_ref[...] + bnb_ref[...]
    z = jnp.maximum(z, 0.0)
    o = jnp.dot(z, wh2_ref[...], preferred_element_type=jnp.float32) + bh2_ref[...]
    out_ref[...] = jnp.clip(o, 0.0, 100.0)


def _head(parts, ln_g, ln_b, Wh1, bh1, bn_g, bn_b, Wh2, bh2):
    return pl.pallas_call(
        _head_body,
        out_shape=jax.ShapeDtypeStruct((_G, 1), jnp.float32),
    )(parts, ln_g, ln_b, Wh1, bh1, bn_g, bn_b, Wh2, bh2)


def kernel(x, edge_index, edge_attr, node_graph_ids, W_e, W1, b1, U1, W2, b2,
           U2, ln_g, ln_b, Wh1, bh1, bn_g, bn_b, Wh2, bh2):
    src = edge_index[0]
    dst = edge_index[1]
    b1r = b1.reshape(1, _D)
    b2r = b2.reshape(1, _D)

    e = _edge_e(edge_attr, W_e)

    hs1 = _sc_gather(x, src)
    m1 = _edge_m(hs1, e, W1, b1r)
    agg1 = _sc_scatter(m1, dst)
    h1 = _node_upd(x, agg1, U1)

    hs2 = _sc_gather(h1, src)
    m2 = _edge_m(hs2, e, W2, b2r)
    agg2 = _sc_scatter(m2, dst)
    h2 = _node_upd(h1, agg2, U2)

    parts = _sc_pool(h2, node_graph_ids)
    out = _head(parts, ln_g.reshape(1, _D), ln_b.reshape(1, _D),
                Wh1, bh1.reshape(1, _D), bn_g.reshape(1, _D),
                bn_b.reshape(1, _D), Wh2, bh2.reshape(1, 1))
    return out[:, 0]


# speedup vs baseline: 1.1917x; 1.0209x over previous
import dataclasses
import functools

import jax
import jax.numpy as jnp
from jax import lax
from jax.experimental import pallas as pl
from jax.experimental.pallas import tpu as pltpu
from jax.experimental.pallas import tpu_sc as plsc

_N = 10000
_E = 320000
_D = 128
_DE = 16
_G = 128

_NC = 2
_NS = 16
_NW = _NC * _NS
_L = 16

_CP = pltpu.CompilerParams()
if "needs_layout_passes" in pltpu.CompilerParams.__dataclass_fields__:
    _CP = dataclasses.replace(_CP, needs_layout_passes=False)

_EPW = _E // _NW
_EB = 80
_NP = 10240
_RPS = _NP // _NS
_ZB = 128

_BE = 2000
_BN = 2000


def _edge_e_body(ea_ref, we_ref, o_ref):
    o_ref[...] = jnp.dot(ea_ref[...], we_ref[...],
                         preferred_element_type=jnp.float32)


def _edge_e(edge_attr, W_e):
    return pl.pallas_call(
        _edge_e_body,
        grid=(_E // _BE,),
        in_specs=[
            pl.BlockSpec((_BE, _DE), lambda i: (i, 0)),
            pl.BlockSpec((_DE, _D), lambda i: (0, 0)),
        ],
        out_specs=pl.BlockSpec((_BE, _D), lambda i: (i, 0)),
        out_shape=jax.ShapeDtypeStruct((_E, _D), jnp.float32),
    )(edge_attr, W_e)


def _edge_m_body(hs_ref, e_ref, w_ref, b_ref, o_ref):
    s = hs_ref[...] + e_ref[...]
    o_ref[...] = jnp.maximum(
        jnp.dot(s, w_ref[...], preferred_element_type=jnp.float32)
        + b_ref[...], 0.0)


def _edge_m(hs, e, W, b2d):
    return pl.pallas_call(
        _edge_m_body,
        grid=(_E // _BE,),
        in_specs=[
            pl.BlockSpec((_BE, _D), lambda i: (i, 0)),
            pl.BlockSpec((_BE, _D), lambda i: (i, 0)),
            pl.BlockSpec((_D, _D), lambda i: (0, 0)),
            pl.BlockSpec((1, _D), lambda i: (0, 0)),
        ],
        out_specs=pl.BlockSpec((_BE, _D), lambda i: (i, 0)),
        out_shape=jax.ShapeDtypeStruct((_E, _D), jnp.float32),
    )(hs, e, W, b2d)


def _node_upd_body(h_ref, agg_ref, u_ref, o_ref):
    o_ref[...] = h_ref[...] + jnp.dot(
        agg_ref[...], u_ref[...], preferred_element_type=jnp.float32)


def _node_upd(h, agg, U):
    return pl.pallas_call(
        _node_upd_body,
        grid=(_N // _BN,),
        in_specs=[
            pl.BlockSpec((_BN, _D), lambda i: (i, 0)),
            pl.BlockSpec((_BN, _D), lambda i: (i, 0)),
            pl.BlockSpec((_D, _D), lambda i: (0, 0)),
        ],
        out_specs=pl.BlockSpec((_BN, _D), lambda i: (i, 0)),
        out_shape=jax.ShapeDtypeStruct((_N, _D), jnp.float32),
    )(h, agg, U)


def _final_body(h_ref, agg_ref, u_ref, ids_ref, lng_ref, lnb_ref, wh1_ref,
                bh1_ref, bng_ref, bnb_ref, wh2_ref, bh2_ref, out_ref,
                pool_ref):
    i = pl.program_id(0)

    @pl.when(i == 0)
    def _():
        pool_ref[...] = jnp.zeros_like(pool_ref)

    h2 = h_ref[...] + jnp.dot(
        agg_ref[...], u_ref[...], preferred_element_type=jnp.float32)
    ids = ids_ref[0]
    gids = lax.broadcasted_iota(jnp.int32, (_G, 1), 0)
    mask = (ids == gids).astype(jnp.float32)
    pool_ref[...] += jnp.dot(mask, h2, preferred_element_type=jnp.float32,
                             precision=lax.Precision.HIGHEST)

    @pl.when(i == pl.num_programs(0) - 1)
    def _():
        g = pool_ref[...]
        mu = jnp.mean(g, axis=-1, keepdims=True)
        var = jnp.mean((g - mu) ** 2, axis=-1, keepdims=True)
        z = (g - mu) * lax.rsqrt(var + 1e-5) * lng_ref[...] + lnb_ref[...]
        z = jnp.dot(z, wh1_ref[...], preferred_element_type=jnp.float32) + bh1_ref[...]
        bmu = jnp.mean(z, axis=0, keepdims=True)
        bvar = jnp.mean((z - bmu) ** 2, axis=0, keepdims=True)
        z = (z - bmu) * lax.rsqrt(bvar + 1e-5) * bng_ref[...] + bnb_ref[...]
        z = jnp.maximum(z, 0.0)
        o = jnp.dot(z, wh2_ref[...], preferred_element_type=jnp.float32) + bh2_ref[...]
        out_ref[...] = jnp.clip(o, 0.0, 100.0)


def _final(h1, agg, U2, ids3d, ln_g, ln_b, Wh1, bh1, bn_g, bn_b, Wh2, bh2):
    out, _ = pl.pallas_call(
        _final_body,
        grid=(_N // _BN,),
        in_specs=[
            pl.BlockSpec((_BN, _D), lambda i: (i, 0)),
            pl.BlockSpec((_BN, _D), lambda i: (i, 0)),
            pl.BlockSpec((_D, _D), lambda i: (0, 0)),
            pl.BlockSpec((1, 1, _BN), lambda i: (i, 0, 0)),
            pl.BlockSpec((1, _D), lambda i: (0, 0)),
            pl.BlockSpec((1, _D), lambda i: (0, 0)),
            pl.BlockSpec((_D, _D), lambda i: (0, 0)),
            pl.BlockSpec((1, _D), lambda i: (0, 0)),
            pl.BlockSpec((1, _D), lambda i: (0, 0)),
            pl.BlockSpec((1, _D), lambda i: (0, 0)),
            pl.BlockSpec((_D, 1), lambda i: (0, 0)),
            pl.BlockSpec((1, 1), lambda i: (0, 0)),
        ],
        out_specs=[
            pl.BlockSpec((_G, 1), lambda i: (0, 0)),
            pl.BlockSpec((_G, _D), lambda i: (0, 0)),
        ],
        out_shape=[
            jax.ShapeDtypeStruct((_G, 1), jnp.float32),
            jax.ShapeDtypeStruct((_G, _D), jnp.float32),
        ],
    )(h1, agg, U2, ids3d, ln_g, ln_b, Wh1, bh1, bn_g, bn_b, Wh2, bh2)
    return out


def _sc_gather_impl(h_hbm, src_hbm, out_hbm, idx_v, rowsA_v, rowsB_v,
                    semA, semB):
    cid = lax.axis_index("c")
    sid = lax.axis_index("s")
    wid = sid * _NC + cid
    base_w = wid * _EPW
    nblk = _EPW // _EB

    pltpu.sync_copy(src_hbm.at[pl.ds(base_w, _EPW)], idx_v)

    def _gstart(k, buf, sem):
        pltpu.async_copy(h_hbm.at[idx_v.at[pl.ds(k * _EB, _EB)]], buf, sem)

    def _gwait(k, buf, sem):
        pltpu.make_async_copy(h_hbm.at[idx_v.at[pl.ds(k * _EB, _EB)]],
                              buf, sem).wait()

    _gstart(0, rowsA_v, semA)

    @pl.loop(0, nblk)
    def _(k):
        @pl.when(k % 2 == 0)
        def _():
            _gwait(k, rowsA_v, semA)

            @pl.when(k + 1 < nblk)
            def _():
                _gstart(k + 1, rowsB_v, semB)
            pltpu.sync_copy(rowsA_v, out_hbm.at[pl.ds(base_w + k * _EB, _EB)])

        @pl.when(k % 2 == 1)
        def _():
            _gwait(k, rowsB_v, semB)

            @pl.when(k + 1 < nblk)
            def _():
                _gstart(k + 1, rowsA_v, semA)
            pltpu.sync_copy(rowsB_v, out_hbm.at[pl.ds(base_w + k * _EB, _EB)])


@functools.cache
def _make_sc_gather():
    return pl.kernel(
        _sc_gather_impl,
        out_type=jax.ShapeDtypeStruct((_E, _D), jnp.float32),
        mesh=plsc.VectorSubcoreMesh(core_axis_name="c", subcore_axis_name="s"),
        compiler_params=_CP,
        scratch_types=[
            pltpu.VMEM((_EPW,), jnp.int32),
            pltpu.VMEM((_EB, _D), jnp.float32),
            pltpu.VMEM((_EB, _D), jnp.float32),
            pltpu.SemaphoreType.DMA,
            pltpu.SemaphoreType.DMA,
        ],
    )


def _sc_gather(h, src):
    return _make_sc_gather()(h, src)


_RPW = _NP // _NW
_SB = 2000
_CAP = 16384
_PB = 128


def _sc_scatter_impl(m_hbm, dst_hbm, out_hbm, blkA_v, blkB_v, idxl_v, dstl_v,
                     bufA_v, bufB_v, acc_v, pos_s, semSA, semSB, semA, semB):
    cid = lax.axis_index("c")
    sid = lax.axis_index("s")
    wid = sid * _NC + cid
    lo = wid * _RPW

    @pl.loop(0, _RPW)
    def _(r):
        @pl.loop(0, _D, step=_L)
        def _(cc):
            acc_v.at[r, pl.ds(cc, _L)][...] = jnp.zeros((_L,), jnp.float32)

    pos_s[0] = 0

    nblk = _E // _SB

    def _scan_block(eb, blk_v):
        @pl.loop(0, _SB, step=2 * _L)
        def _(c0):
            d0 = blk_v[pl.ds(c0, _L)]
            d1 = blk_v[pl.ds(c0 + _L, _L)]
            msk0 = (d0 >= lo) & (d0 < lo + _RPW)
            msk1 = (d1 >= lo) & (d1 < lo + _RPW)
            mi0 = msk0.astype(jnp.int32)
            mi1 = msk1.astype(jnp.int32)
            cnt0 = jnp.sum(mi0)
            cnt1 = jnp.sum(mi1)

            @pl.when(cnt0 + cnt1 > 0)
            def _():
                pos = pos_s[0]
                pref0 = plsc.cumsum(mi0)
                pref1 = plsc.cumsum(mi1)
                positions0 = pos + pref0 - 1
                positions1 = pos + cnt0 + pref1 - 1
                iot = lax.iota(jnp.int32, _L)
                plsc.store_scatter(idxl_v, [positions0], eb + c0 + iot,
                                   mask=msk0)
                plsc.store_scatter(dstl_v, [positions0], d0 - lo, mask=msk0)
                plsc.store_scatter(idxl_v, [positions1], eb + c0 + _L + iot,
                                   mask=msk1)
                plsc.store_scatter(dstl_v, [positions1], d1 - lo, mask=msk1)
                pos_s[0] = pos + cnt0 + cnt1

    pltpu.async_copy(dst_hbm.at[pl.ds(0, _SB)], blkA_v, semSA)

    @pl.loop(0, nblk)
    def _(k):
        @pl.when(k % 2 == 0)
        def _():
            pltpu.make_async_copy(dst_hbm.at[pl.ds(k * _SB, _SB)],
                                  blkA_v, semSA).wait()

            @pl.when(k + 1 < nblk)
            def _():
                pltpu.async_copy(dst_hbm.at[pl.ds((k + 1) * _SB, _SB)],
                                 blkB_v, semSB)
            _scan_block(k * _SB, blkA_v)

        @pl.when(k % 2 == 1)
        def _():
            pltpu.make_async_copy(dst_hbm.at[pl.ds(k * _SB, _SB)],
                                  blkB_v, semSB).wait()

            @pl.when(k + 1 < nblk)
            def _():
                pltpu.async_copy(dst_hbm.at[pl.ds((k + 1) * _SB, _SB)],
                                 blkA_v, semSA)
            _scan_block(k * _SB, blkB_v)

    total = pos_s[0]

    @pl.loop(0, _PB, step=_L)
    def _(t):
        plsc.store_scatter(idxl_v, [total + t + lax.iota(jnp.int32, _L)],
                           jnp.zeros((_L,), jnp.int32))

    nb = (total + _PB - 1) // _PB

    def _gather_start(j, buf_v, sem):
        pltpu.async_copy(m_hbm.at[idxl_v.at[pl.ds(j * _PB, _PB)]], buf_v, sem)

    def _gather_wait(j, buf_v, sem):
        pltpu.make_async_copy(m_hbm.at[idxl_v.at[pl.ds(j * _PB, _PB)]],
                              buf_v, sem).wait()

    def _apply(j, buf_v):
        @pl.loop(0, _PB, step=_L)
        def _(b0):
            for l in range(_L):
                valid = jnp.broadcast_to(j * _PB + b0 + l < total, (_L,))
                dsplat = plsc.load_gather(
                    dstl_v, [jnp.broadcast_to(j * _PB + b0 + l, (_L,))])
                for c in range(_D // _L):
                    col = lax.iota(jnp.int32, _L) + c * _L
                    v = buf_v[b0 + l, pl.ds(c * _L, _L)]
                    plsc.addupdate_scatter(acc_v, [dsplat, col], v, mask=valid)

    @pl.when(nb > 0)
    def _():
        _gather_start(0, bufA_v, semA)

    @pl.loop(0, nb)
    def _(j):
        @pl.when(j % 2 == 0)
        def _():
            _gather_wait(j, bufA_v, semA)

            @pl.when(j + 1 < nb)
            def _():
                _gather_start(j + 1, bufB_v, semB)
            _apply(j, bufA_v)

        @pl.when(j % 2 == 1)
        def _():
            _gather_wait(j, bufB_v, semB)

            @pl.when(j + 1 < nb)
            def _():
                _gather_start(j + 1, bufA_v, semA)
            _apply(j, bufB_v)

    pltpu.sync_copy(acc_v, out_hbm.at[pl.ds(lo, _RPW)])


@functools.cache
def _make_sc_scatter():
    return pl.kernel(
        _sc_scatter_impl,
        out_type=jax.ShapeDtypeStruct((_NP, _D), jnp.float32),
        mesh=plsc.VectorSubcoreMesh(core_axis_name="c", subcore_axis_name="s"),
        compiler_params=_CP,
        scratch_types=[
            pltpu.VMEM((_SB,), jnp.int32),
            pltpu.VMEM((_SB,), jnp.int32),
            pltpu.VMEM((_CAP,), jnp.int32),
            pltpu.VMEM((_CAP,), jnp.int32),
            pltpu.VMEM((_PB, _D), jnp.float32),
            pltpu.VMEM((_PB, _D), jnp.float32),
            pltpu.VMEM((_RPW, _D), jnp.float32),
            pltpu.SMEM((1,), jnp.int32),
            pltpu.SemaphoreType.DMA,
            pltpu.SemaphoreType.DMA,
            pltpu.SemaphoreType.DMA,
            pltpu.SemaphoreType.DMA,
        ],
    )


def _sc_scatter(m, dst):
    return _make_sc_scatter()(m, dst)


_PBND = (0, 720, 1440, 2160, 2880, 3600, 4320, 5040, 5760, 6480, 7200,
         7680, 8160, 8640, 9120, 9600, 10000)
_NPC = len(_PBND) - 1
_PRB = 80


def _sc_pool_impl(h_hbm, ids_hbm, out_hbm, ids_v, rows_v, acc_v, sem):
    cid = lax.axis_index("c")
    sid = lax.axis_index("s")
    wid = sid * _NC + cid

    @pl.when(wid < _NPC)
    def _():
        lo80 = jnp.int32(0)
        hi80 = jnp.int32(0)
        for j in range(_NPC):
            lo80 = jnp.where(wid == j, jnp.int32(_PBND[j] // _PRB), lo80)
            hi80 = jnp.where(wid == j, jnp.int32(_PBND[j + 1] // _PRB), hi80)

        @pl.loop(0, _G)
        def _(r):
            @pl.loop(0, _D, step=_L)
            def _(cc):
                acc_v.at[r, pl.ds(cc, _L)][...] = jnp.zeros(
                    (_L,), jnp.float32)

        nblk = hi80 - lo80

        @pl.loop(0, nblk)
        def _(k):
            base = (lo80 + k) * _PRB
            pltpu.sync_copy(ids_hbm.at[pl.ds(base, _PRB)], ids_v)
            pltpu.sync_copy(h_hbm.at[pl.ds(base, _PRB)], rows_v)

            @pl.loop(0, _PRB)
            def _(r):
                gsplat = plsc.load_gather(ids_v, [jnp.broadcast_to(r, (_L,))])
                for c in range(_D // _L):
                    col = lax.iota(jnp.int32, _L) + c * _L
                    v = rows_v[r, pl.ds(c * _L, _L)]
                    plsc.addupdate_scatter(acc_v, [gsplat, col], v)

        pltpu.sync_copy(acc_v, out_hbm.at[wid])


@functools.cache
def _make_sc_pool():
    return pl.kernel(
        _sc_pool_impl,
        out_type=jax.ShapeDtypeStruct((_NPC, _G, _D), jnp.float32),
        mesh=plsc.VectorSubcoreMesh(core_axis_name="c", subcore_axis_name="s"),
        compiler_params=_CP,
        scratch_types=[
            pltpu.VMEM((_PRB,), jnp.int32),
            pltpu.VMEM((_PRB, _D), jnp.float32),
            pltpu.VMEM((_G, _D), jnp.float32),
            pltpu.SemaphoreType.DMA,
        ],
    )


def _sc_pool(h2, ids):
    return _make_sc_pool()(h2, ids)


def _head_body(parts_ref, lng_ref, lnb_ref, wh1_ref, bh1_ref, bng_ref,
               bnb_ref, wh2_ref, bh2_ref, out_ref):
    g = parts_ref[0]
    for w in range(1, _NPC):
        g = g + parts_ref[w]
    mu = jnp.mean(g, axis=-1, keepdims=True)
    var = jnp.mean((g - mu) ** 2, axis=-1, keepdims=True)
    z = (g - mu) * lax.rsqrt(var + 1e-5) * lng_ref[...] + lnb_ref[...]
    z = jnp.dot(z, wh1_ref[...], preferred_element_type=jnp.float32) + bh1_ref[...]
    bmu = jnp.mean(z, axis=0, keepdims=True)
    bvar = jnp.mean((z - bmu) ** 2, axis=0, keepdims=True)
    z = (z - bmu) * lax.rsqrt(bvar + 1e-5) * bng_ref[...] + bnb_ref[...]
    z = jnp.maximum(z, 0.0)
    o = jnp.dot(z, wh2_ref[...], preferred_element_type=jnp.float32) + bh2_ref[...]
    out_ref[...] = jnp.clip(o, 0.0, 100.0)


def _head(parts, ln_g, ln_b, Wh1, bh1, bn_g, bn_b, Wh2, bh2):
    return pl.pallas_call(
        _head_body,
        out_shape=jax.ShapeDtypeStruct((_G, 1), jnp.float32),
    )(parts, ln_g, ln_b, Wh1, bh1, bn_g, bn_b, Wh2, bh2)


def kernel(x, edge_index, edge_attr, node_graph_ids, W_e, W1, b1, U1, W2, b2,
           U2, ln_g, ln_b, Wh1, bh1, bn_g, bn_b, Wh2, bh2):
    src = edge_index[0]
    dst = edge_index[1]
    b1r = b1.reshape(1, _D)
    b2r = b2.reshape(1, _D)

    e = _edge_e(edge_attr, W_e)

    hs1 = _sc_gather(x, src)
    m1 = _edge_m(hs1, e, W1, b1r)
    agg1 = _sc_scatter(m1, dst)
    h1 = _node_upd(x, agg1, U1)

    hs2 = _sc_gather(h1, src)
    m2 = _edge_m(hs2, e, W2, b2r)
    agg2 = _sc_scatter(m2, dst)
    h2 = _node_upd(h1, agg2, U2)

    parts = _sc_pool(h2, node_graph_ids)
    out = _head(parts, ln_g.reshape(1, _D), ln_b.reshape(1, _D),
                Wh1, bh1.reshape(1, _D), bn_g.reshape(1, _D),
                bn_b.reshape(1, _D), Wh2, bh2.reshape(1, 1))
    return out[:, 0]
